# Initial kernel scaffold; baseline (speedup 1.0000x reference)
import jax
import jax.numpy as jnp
from jax.experimental import pallas as pl


def kernel(x_s, edge_index_s, x_t, edge_index_t, y, W1, b1, W2, b2, fcW, fcb, fc2W, fc2b):
    raise NotImplementedError("write your pallas kernel here")


# speedup vs baseline: 7.5273x; 7.5273x over previous
import functools

import jax
import jax.numpy as jnp
from jax import lax
from jax.experimental import pallas as pl
from jax.experimental.pallas import tpu as pltpu
from jax.experimental.pallas import tpu_sc as plsc

N = 10000
NP = 10240
NG = 2
E = 320000
D = 128
NS = 16
CHUNK = 80
EPT = E // NS
NCH = EPT // CHUNK
RPT = NP // NS
RCH = 128
P2 = 102400
PPW = P2 // (NG * NS)
PCH = PPW // CHUNK


@functools.cache
def _deg_kernel():
    mesh = plsc.VectorSubcoreMesh(core_axis_name="c", subcore_axis_name="s")
    return pl.kernel(
        _deg_body,
        out_type=jax.ShapeDtypeStruct((NG * NS * 8, D), jnp.float32),
        mesh=mesh,
        scratch_types=[
            pltpu.VMEM((CHUNK,), jnp.int32),
            pltpu.VMEM((CHUNK, D), jnp.float32),
            pltpu.VMEM((RCH, D), jnp.float32),
            pltpu.VMEM((8, D), jnp.float32),
            pltpu.VMEM_SHARED((NP, D), jnp.float32),
        ],
    )


def _deg_body(dst_hbm, ones_hbm, zeros_hbm, out_hbm, didx, ones_v, zb, zb1, acc):
    c = lax.axis_index("c")
    s = lax.axis_index("s")
    pltpu.sync_copy(ones_hbm, ones_v)
    pltpu.sync_copy(zeros_hbm, zb)
    for k in range(RPT // RCH):
        pltpu.sync_copy(zb, acc.at[pl.ds(s * RPT + k * RCH, RCH)])
    plsc.subcore_barrier()

    def body(i, carry):
        base = c * E + s * EPT + i * CHUNK
        pltpu.sync_copy(dst_hbm.at[pl.ds(base, CHUNK)], didx)
        pltpu.sync_copy(ones_v, acc.at[didx], add=True)
        return carry

    lax.fori_loop(0, NCH, body, 0)
    plsc.subcore_barrier()

    lane = lax.iota(jnp.int32, 16)
    w = c * NS + s
    for k in range(RPT // RCH):
        pltpu.sync_copy(acc.at[pl.ds(s * RPT + k * RCH, RCH)], zb)

        def extract(i, carry):
            vec = jnp.zeros((16,), jnp.float32)
            for r in range(16):
                row = zb[i * 16 + r, pl.ds(0, 16)]
                vec = jnp.where(lane == r, row, vec)
            zb1[k, pl.ds(i * 16, 16)] = vec
            return carry

        lax.fori_loop(0, RCH // 16, extract, 0)
    pltpu.sync_copy(zb1, out_hbm.at[pl.ds(w * 8, 8)])


@functools.cache
def _agg_kernel():
    mesh = plsc.VectorSubcoreMesh(core_axis_name="c", subcore_axis_name="s")
    return pl.kernel(
        _agg_body,
        out_type=jax.ShapeDtypeStruct((NG * NP, D), jnp.float32),
        mesh=mesh,
        scratch_types=[
            pltpu.VMEM((CHUNK,), jnp.int32),
            pltpu.VMEM((CHUNK,), jnp.int32),
            pltpu.VMEM((CHUNK, D), jnp.float32),
            pltpu.VMEM((RCH, D), jnp.float32),
            pltpu.VMEM_SHARED((NP, D), jnp.float32),
            pltpu.SemaphoreType.DMA,
        ],
    )


def _agg_body(h_hbm, src_hbm, dst_hbm, zeros_hbm, out_hbm,
              sidx, didx, rows, zb, acc, sem):
    c = lax.axis_index("c")
    s = lax.axis_index("s")
    pltpu.sync_copy(zeros_hbm, zb)
    for k in range(RPT // RCH):
        pltpu.sync_copy(zb, acc.at[pl.ds(s * RPT + k * RCH, RCH)])
    plsc.subcore_barrier()

    def body(i, carry):
        base = c * E + s * EPT + i * CHUNK
        pltpu.sync_copy(src_hbm.at[pl.ds(base, CHUNK)], sidx)
        pltpu.sync_copy(dst_hbm.at[pl.ds(base, CHUNK)], didx)
        pltpu.async_copy(h_hbm.at[sidx], rows, sem).wait()
        pltpu.sync_copy(rows, acc.at[didx], add=True)
        return carry

    lax.fori_loop(0, NCH, body, 0)
    plsc.subcore_barrier()
    for k in range(RPT // RCH):
        pltpu.sync_copy(acc.at[pl.ds(s * RPT + k * RCH, RCH)], zb)
        pltpu.sync_copy(zb, out_hbm.at[pl.ds(c * NP + s * RPT + k * RCH, RCH)])


@functools.cache
def _pair_kernel():
    mesh = plsc.VectorSubcoreMesh(core_axis_name="c", subcore_axis_name="s")
    return pl.kernel(
        _pair_body,
        out_type=jax.ShapeDtypeStruct((P2, D), jnp.float32),
        mesh=mesh,
        scratch_types=[
            pltpu.VMEM((CHUNK,), jnp.int32),
            pltpu.VMEM((CHUNK,), jnp.int32),
            pltpu.VMEM((CHUNK, D), jnp.float32),
            pltpu.VMEM((CHUNK, D), jnp.float32),
            pltpu.SemaphoreType.DMA,
        ],
    )


def _pair_body(g_hbm, y0_hbm, y1_hbm, out_hbm, i0, i1, ba, bb, sem):
    c = lax.axis_index("c")
    s = lax.axis_index("s")
    w = s * NG + c

    def body(i, carry):
        base = w * PPW + i * CHUNK
        pltpu.sync_copy(y0_hbm.at[pl.ds(base, CHUNK)], i0)
        pltpu.sync_copy(y1_hbm.at[pl.ds(base, CHUNK)], i1)
        pltpu.async_copy(g_hbm.at[i0], ba, sem).wait()
        pltpu.async_copy(g_hbm.at[i1], bb, sem).wait()

        def add_row(r, rc):
            for j in range(D // 16):
                sl = pl.ds(j * 16, 16)
                ba[r, sl] = ba[r, sl] + bb[r, sl]
            return rc

        lax.fori_loop(0, CHUNK, add_row, 0)
        pltpu.sync_copy(ba, out_hbm.at[pl.ds(base, CHUNK)])
        return carry

    lax.fori_loop(0, PCH, body, 0)


_BR = 2048
_BP = 2048

_HI = lax.Precision.HIGHEST


def _mm1_body(x_ref, deg_ref, w_ref, o_ref):
    dis = lax.rsqrt(deg_ref[:, 0] + 1.0)
    h = jnp.dot(x_ref[...], w_ref[...], preferred_element_type=jnp.float32,
                precision=_HI)
    o_ref[...] = h * dis[:, None]


def _mm1(xx, deg16, W1):
    return pl.pallas_call(
        _mm1_body,
        out_shape=jax.ShapeDtypeStruct((NG * NP, D), jnp.float32),
        grid=(NG * NP // _BR,),
        in_specs=[pl.BlockSpec((_BR, D), lambda i: (i, 0)),
                  pl.BlockSpec((_BR, 1), lambda i: (i, 0)),
                  pl.BlockSpec((D, D), lambda i: (0, 0))],
        out_specs=pl.BlockSpec((_BR, D), lambda i: (i, 0)),
    )(xx, deg16, W1)


def _mm2_body(agg_ref, hp_ref, deg_ref, w_ref, b_ref, o_ref):
    dis = lax.rsqrt(deg_ref[:, 0] + 1.0)
    x2 = jnp.maximum((agg_ref[...] + hp_ref[...]) * dis[:, None] + b_ref[...], 0.0)
    h = jnp.dot(x2, w_ref[...], preferred_element_type=jnp.float32, precision=_HI)
    o_ref[...] = h * dis[:, None]


def _mm2(agg, hp, deg16, W2, b1):
    return pl.pallas_call(
        _mm2_body,
        out_shape=jax.ShapeDtypeStruct((NG * NP, D), jnp.float32),
        grid=(NG * NP // _BR,),
        in_specs=[pl.BlockSpec((_BR, D), lambda i: (i, 0)),
                  pl.BlockSpec((_BR, D), lambda i: (i, 0)),
                  pl.BlockSpec((_BR, 1), lambda i: (i, 0)),
                  pl.BlockSpec((D, D), lambda i: (0, 0)),
                  pl.BlockSpec((1, D), lambda i: (0, 0))],
        out_specs=pl.BlockSpec((_BR, D), lambda i: (i, 0)),
    )(agg, hp, deg16, W2, b1)


def _mm3_body(agg_ref, hp_ref, deg_ref, b_ref, fw_ref, o_ref):
    dis = lax.rsqrt(deg_ref[:, 0] + 1.0)
    xf = jnp.maximum((agg_ref[...] + hp_ref[...]) * dis[:, None] + b_ref[...], 0.0)
    o_ref[...] = jnp.dot(xf, fw_ref[0], preferred_element_type=jnp.float32,
                         precision=_HI)


def _mm3(agg, hp, deg16, b2, fcW3):
    nblk = NG * NP // _BR
    half = nblk // NG
    return pl.pallas_call(
        _mm3_body,
        out_shape=jax.ShapeDtypeStruct((NG * NP, D), jnp.float32),
        grid=(nblk,),
        in_specs=[pl.BlockSpec((_BR, D), lambda i: (i, 0)),
                  pl.BlockSpec((_BR, D), lambda i: (i, 0)),
                  pl.BlockSpec((_BR, 1), lambda i: (i, 0)),
                  pl.BlockSpec((1, D), lambda i: (0, 0)),
                  pl.BlockSpec((1, D, D), lambda i: (i // half, 0, 0))],
        out_specs=pl.BlockSpec((_BR, D), lambda i: (i, 0)),
    )(agg, hp, deg16, b2, fcW3)


def _fc2_body(s_ref, fcb_ref, w_ref, b2_ref, o_ref):
    z = jnp.maximum(s_ref[...] + fcb_ref[...], 0.0)
    o = jnp.dot(z, w_ref[...], preferred_element_type=jnp.float32,
                precision=_HI) + b2_ref[...]
    o_ref[...] = 1.0 / (1.0 + jnp.exp(-o))


def _fc2(S, fcb, fc2W, fc2b):
    return pl.pallas_call(
        _fc2_body,
        out_shape=jax.ShapeDtypeStruct((P2, 1), jnp.float32),
        grid=(P2 // _BP,),
        in_specs=[pl.BlockSpec((_BP, D), lambda i: (i, 0)),
                  pl.BlockSpec((1, D), lambda i: (0, 0)),
                  pl.BlockSpec((D, 1), lambda i: (0, 0)),
                  pl.BlockSpec((1, 1), lambda i: (0, 0))],
        out_specs=pl.BlockSpec((_BP, 1), lambda i: (i, 0)),
    )(S, fcb, fc2W, fc2b)


def kernel(x_s, edge_index_s, x_t, edge_index_t, y, W1, b1, W2, b2,
           fcW, fcb, fc2W, fc2b):
    P = y.shape[0]
    zpad = jnp.zeros((NP - N, D), jnp.float32)
    xx = jnp.concatenate([x_s, zpad, x_t, zpad], axis=0)
    src = jnp.concatenate([edge_index_s[0].astype(jnp.int32),
                           edge_index_t[0].astype(jnp.int32) + NP])
    dst = jnp.concatenate([edge_index_s[1].astype(jnp.int32),
                           edge_index_t[1].astype(jnp.int32)])
    pad = P2 - P
    y0 = jnp.concatenate([y[:, 0].astype(jnp.int32),
                          jnp.zeros((pad,), jnp.int32)])
    y1 = jnp.concatenate([y[:, 1].astype(jnp.int32) + NP,
                          jnp.full((pad,), NP, jnp.int32)])
    zeros128 = jnp.zeros((RCH, D), jnp.float32)

    ones128 = jnp.ones((CHUNK, D), jnp.float32)
    degp = _deg_kernel()(dst, ones128, zeros128)
    deg16 = degp.reshape(NG * NS, 8, D)[:, :RPT // RCH, :].reshape(NG * NP, 1)
    h1p = _mm1(xx, deg16, W1)
    agg1 = _agg_kernel()(h1p, src, dst, zeros128)
    h2p = _mm2(agg1, h1p, deg16, W2, b1.reshape(1, D))
    agg2 = _agg_kernel()(h2p, src, dst, zeros128)
    G = _mm3(agg2, h2p, deg16, b2.reshape(1, D), fcW.reshape(NG, D, D))
    S = _pair_kernel()(G, y0, y1)
    out = _fc2(S, fcb.reshape(1, D), fc2W, fc2b.reshape(1, 1))
    return out[:P]


# speedup vs baseline: 15.1614x; 2.0142x over previous
import functools

import jax
import jax.numpy as jnp
from jax import lax
from jax.experimental import pallas as pl
from jax.experimental.pallas import tpu as pltpu
from jax.experimental.pallas import tpu_sc as plsc

N = 10000
NP = 10240
NG = 2
E = 320000
E2 = 327680
D = 128
NS = 16
CHUNK = 128
EPT = E2 // NS
NCH = EPT // CHUNK
RPT = NP // NS
RCH = 128
NB = 2
GS = 40
NGR = NCH // GS
P2 = 102400
PPW = P2 // (NG * NS)
PCH = PPW // CHUNK


@functools.cache
def _deg_kernel():
    mesh = plsc.VectorSubcoreMesh(core_axis_name="c", subcore_axis_name="s")
    return pl.kernel(
        _deg_body,
        out_type=jax.ShapeDtypeStruct((NG * NS * 8, D), jnp.float32),
        mesh=mesh,
        scratch_types=[
            pltpu.VMEM((NCH, CHUNK), jnp.int32),
            pltpu.VMEM((RCH, D), jnp.float32),
            pltpu.VMEM((8, D), jnp.float32),
            pltpu.VMEM_SHARED((NP, D), jnp.float32),
            pltpu.SemaphoreType.DMA,
        ],
    )


def _deg_body(dst_hbm, ones_hbm, zeros_hbm, out_hbm,
              didx, zb, zb1, acc, ssem):
    c = lax.axis_index("c")
    s = lax.axis_index("s")
    w = c * NS + s
    pltpu.sync_copy(dst_hbm.at[pl.ds(w * NCH, NCH)], didx)
    pltpu.sync_copy(zeros_hbm, zb)
    for k in range(RPT // RCH):
        pltpu.sync_copy(zb, acc.at[pl.ds(s * RPT + k * RCH, RCH)])
    plsc.subcore_barrier()
    pltpu.sync_copy(ones_hbm, zb)

    def fire(j):
        pltpu.async_copy(zb, acc.at[didx.at[j]], ssem, add=True)

    def drain():
        pltpu.make_async_copy(ones_hbm, zb, ssem).wait()

    for j in range(8):
        fire(j)

    def body(i, carry):
        drain()
        fire(i)
        return carry

    lax.fori_loop(8, NCH, body, 0)
    for _ in range(8):
        drain()
    plsc.subcore_barrier()

    lane = lax.iota(jnp.int32, 16)
    for k in range(RPT // RCH):
        pltpu.sync_copy(acc.at[pl.ds(s * RPT + k * RCH, RCH)], zb)

        def extract(i, carry):
            vec = jnp.zeros((16,), jnp.float32)
            for r in range(16):
                row = zb[i * 16 + r, pl.ds(0, 16)]
                vec = jnp.where(lane == r, row, vec)
            zb1[k, pl.ds(i * 16, 16)] = vec
            return carry

        lax.fori_loop(0, RCH // 16, extract, 0)
    pltpu.sync_copy(zb1, out_hbm.at[pl.ds(w * 8, 8)])


@functools.cache
def _agg_kernel():
    mesh = plsc.VectorSubcoreMesh(core_axis_name="c", subcore_axis_name="s")
    return pl.kernel(
        _agg_body,
        out_type=jax.ShapeDtypeStruct((NG * NP, D), jnp.float32),
        mesh=mesh,
        scratch_types=[
            pltpu.VMEM((GS, CHUNK), jnp.int32),
            pltpu.VMEM((GS, CHUNK), jnp.int32),
            pltpu.VMEM((NB, CHUNK, D), jnp.float32),
            pltpu.VMEM_SHARED((NP, D), jnp.float32),
            pltpu.SemaphoreType.DMA,
            pltpu.SemaphoreType.DMA,
        ],
    )


def _agg_body(h_hbm, src_hbm, dst_hbm, zeros_hbm, out_hbm,
              sidx, didx, rows, acc, gsem, ssem):
    c = lax.axis_index("c")
    s = lax.axis_index("s")
    w = c * NS + s
    pltpu.sync_copy(zeros_hbm, rows.at[0])
    for k in range(RPT // RCH):
        pltpu.sync_copy(rows.at[0], acc.at[pl.ds(s * RPT + k * RCH, RCH)])
    plsc.subcore_barrier()

    def fire_g(j, b):
        pltpu.async_copy(h_hbm.at[sidx.at[j]], rows.at[b], gsem)

    def fire_s(j, b):
        pltpu.async_copy(rows.at[b], acc.at[didx.at[j]], ssem, add=True)

    def drain(sem):
        pltpu.make_async_copy(h_hbm.at[pl.ds(0, CHUNK)], rows.at[0], sem).wait()

    for g in range(NGR):
        base = w * NCH + g * GS
        pltpu.sync_copy(src_hbm.at[pl.ds(base, GS)], sidx)
        pltpu.sync_copy(dst_hbm.at[pl.ds(base, GS)], didx)
        fire_g(0, 0)
        drain(gsem)
        fire_s(0, 0)
        fire_g(1, 1)

        def body(i, carry):
            b = lax.rem(i, 2)
            drain(gsem)
            fire_s(i, b)
            drain(ssem)
            fire_g(i + 1, 1 - b)
            return carry

        lax.fori_loop(1, GS - 1, body, 0)
        drain(gsem)
        fire_s(GS - 1, (GS - 1) % 2)
        drain(ssem)
        drain(ssem)
    plsc.subcore_barrier()
    for k in range(RPT // RCH):
        pltpu.sync_copy(acc.at[pl.ds(s * RPT + k * RCH, RCH)], rows.at[0])
        pltpu.sync_copy(rows.at[0],
                        out_hbm.at[pl.ds(c * NP + s * RPT + k * RCH, RCH)])


@functools.cache
def _pair_kernel():
    mesh = plsc.VectorSubcoreMesh(core_axis_name="c", subcore_axis_name="s")
    return pl.kernel(
        _pair_body,
        out_type=jax.ShapeDtypeStruct((P2, D), jnp.float32),
        mesh=mesh,
        scratch_types=[
            pltpu.VMEM((PCH, CHUNK), jnp.int32),
            pltpu.VMEM((PCH, CHUNK), jnp.int32),
            pltpu.VMEM((2, CHUNK, D), jnp.float32),
            pltpu.VMEM((2, CHUNK, D), jnp.float32),
            pltpu.SemaphoreType.DMA,
            pltpu.SemaphoreType.DMA,
        ],
    )


def _pair_body(g_hbm, y0_hbm, y1_hbm, out_hbm, yi0, yi1, ba, bb, gsem, wsem):
    c = lax.axis_index("c")
    s = lax.axis_index("s")
    w = s * NG + c
    pltpu.sync_copy(y0_hbm.at[w], yi0)
    pltpu.sync_copy(y1_hbm.at[w], yi1)

    def fire_g(j, b):
        pltpu.async_copy(g_hbm.at[yi0.at[j]], ba.at[b], gsem)
        pltpu.async_copy(g_hbm.at[yi1.at[j]], bb.at[b], gsem)

    def drain(sem):
        pltpu.make_async_copy(g_hbm.at[pl.ds(0, CHUNK)], ba.at[0], sem).wait()

    def add_write(j, b):
        def add_row(r, rc):
            for jj in range(D // 16):
                sl = pl.ds(jj * 16, 16)
                ba[b, r, sl] = ba[b, r, sl] + bb[b, r, sl]
            return rc

        lax.fori_loop(0, CHUNK, add_row, 0)
        pltpu.async_copy(ba.at[b],
                         out_hbm.at[pl.ds((w * PCH + j) * CHUNK, CHUNK)], wsem)

    fire_g(0, 0)
    drain(gsem)
    drain(gsem)
    fire_g(1, 1)
    add_write(0, 0)

    def body(j, carry):
        b = lax.rem(j, 2)
        drain(gsem)
        drain(gsem)
        drain(wsem)
        fire_g(j + 1, 1 - b)
        add_write(j, b)
        return carry

    lax.fori_loop(1, PCH - 1, body, 0)
    drain(gsem)
    drain(gsem)
    drain(wsem)
    add_write(PCH - 1, (PCH - 1) % 2)
    drain(wsem)


_BR = 2048
_BP = 2048

_HI = lax.Precision.HIGHEST


def _mm1_body(x_ref, deg_ref, w_ref, o_ref):
    dis = lax.rsqrt(deg_ref[:, 0] + 1.0)
    h = jnp.dot(x_ref[...], w_ref[...], preferred_element_type=jnp.float32,
                precision=_HI)
    o_ref[...] = h * dis[:, None]


def _mm1(xx, deg16, W1):
    return pl.pallas_call(
        _mm1_body,
        out_shape=jax.ShapeDtypeStruct((NG * NP, D), jnp.float32),
        grid=(NG * NP // _BR,),
        in_specs=[pl.BlockSpec((_BR, D), lambda i: (i, 0)),
                  pl.BlockSpec((_BR, 1), lambda i: (i, 0)),
                  pl.BlockSpec((D, D), lambda i: (0, 0))],
        out_specs=pl.BlockSpec((_BR, D), lambda i: (i, 0)),
    )(xx, deg16, W1)


def _mm2_body(agg_ref, hp_ref, deg_ref, w_ref, b_ref, o_ref):
    dis = lax.rsqrt(deg_ref[:, 0] + 1.0)
    x2 = jnp.maximum((agg_ref[...] + hp_ref[...]) * dis[:, None] + b_ref[...], 0.0)
    h = jnp.dot(x2, w_ref[...], preferred_element_type=jnp.float32, precision=_HI)
    o_ref[...] = h * dis[:, None]


def _mm2(agg, hp, deg16, W2, b1):
    return pl.pallas_call(
        _mm2_body,
        out_shape=jax.ShapeDtypeStruct((NG * NP, D), jnp.float32),
        grid=(NG * NP // _BR,),
        in_specs=[pl.BlockSpec((_BR, D), lambda i: (i, 0)),
                  pl.BlockSpec((_BR, D), lambda i: (i, 0)),
                  pl.BlockSpec((_BR, 1), lambda i: (i, 0)),
                  pl.BlockSpec((D, D), lambda i: (0, 0)),
                  pl.BlockSpec((1, D), lambda i: (0, 0))],
        out_specs=pl.BlockSpec((_BR, D), lambda i: (i, 0)),
    )(agg, hp, deg16, W2, b1)


def _mm3_body(agg_ref, hp_ref, deg_ref, b_ref, fw_ref, o_ref):
    dis = lax.rsqrt(deg_ref[:, 0] + 1.0)
    xf = jnp.maximum((agg_ref[...] + hp_ref[...]) * dis[:, None] + b_ref[...], 0.0)
    o_ref[...] = jnp.dot(xf, fw_ref[0], preferred_element_type=jnp.float32,
                         precision=_HI)


def _mm3(agg, hp, deg16, b2, fcW3):
    nblk = NG * NP // _BR
    half = nblk // NG
    return pl.pallas_call(
        _mm3_body,
        out_shape=jax.ShapeDtypeStruct((NG * NP, D), jnp.float32),
        grid=(nblk,),
        in_specs=[pl.BlockSpec((_BR, D), lambda i: (i, 0)),
                  pl.BlockSpec((_BR, D), lambda i: (i, 0)),
                  pl.BlockSpec((_BR, 1), lambda i: (i, 0)),
                  pl.BlockSpec((1, D), lambda i: (0, 0)),
                  pl.BlockSpec((1, D, D), lambda i: (i // half, 0, 0))],
        out_specs=pl.BlockSpec((_BR, D), lambda i: (i, 0)),
    )(agg, hp, deg16, b2, fcW3)


def _fc2_body(s_ref, fcb_ref, w_ref, b2_ref, o_ref):
    z = jnp.maximum(s_ref[...] + fcb_ref[...], 0.0)
    o = jnp.dot(z, w_ref[...], preferred_element_type=jnp.float32,
                precision=_HI) + b2_ref[...]
    o_ref[...] = 1.0 / (1.0 + jnp.exp(-o))


def _fc2(S, fcb, fc2W, fc2b):
    return pl.pallas_call(
        _fc2_body,
        out_shape=jax.ShapeDtypeStruct((P2, 1), jnp.float32),
        grid=(P2 // _BP,),
        in_specs=[pl.BlockSpec((_BP, D), lambda i: (i, 0)),
                  pl.BlockSpec((1, D), lambda i: (0, 0)),
                  pl.BlockSpec((D, 1), lambda i: (0, 0)),
                  pl.BlockSpec((1, 1), lambda i: (0, 0))],
        out_specs=pl.BlockSpec((_BP, 1), lambda i: (i, 0)),
    )(S, fcb, fc2W, fc2b)


def kernel(x_s, edge_index_s, x_t, edge_index_t, y, W1, b1, W2, b2,
           fcW, fcb, fc2W, fc2b):
    P = y.shape[0]
    zpad = jnp.zeros((NP - N, D), jnp.float32)
    xx = jnp.concatenate([x_s, zpad, x_t, zpad], axis=0)

    epad = E2 - E
    pad_src = (jnp.arange(epad, dtype=jnp.int32) * 37) % N
    pad_dst = N + (jnp.arange(epad, dtype=jnp.int32) % (NP - N))
    src = jnp.concatenate([edge_index_s[0].astype(jnp.int32), pad_src,
                           edge_index_t[0].astype(jnp.int32) + NP,
                           pad_src + NP]).reshape(NG * NS * NCH, CHUNK)
    dst = jnp.concatenate([edge_index_s[1].astype(jnp.int32), pad_dst,
                           edge_index_t[1].astype(jnp.int32),
                           pad_dst]).reshape(NG * NS * NCH, CHUNK)

    pad = P2 - P
    pad_y = (jnp.arange(pad, dtype=jnp.int32) * 53) % N
    y0 = jnp.concatenate([y[:, 0].astype(jnp.int32),
                          pad_y]).reshape(NG * NS, PCH, CHUNK)
    y1 = jnp.concatenate([y[:, 1].astype(jnp.int32) + NP,
                          pad_y + NP]).reshape(NG * NS, PCH, CHUNK)

    ones128 = jnp.ones((CHUNK, D), jnp.float32)
    zeros128 = jnp.zeros((RCH, D), jnp.float32)

    degp = _deg_kernel()(dst, ones128, zeros128)
    deg16 = degp.reshape(NG * NS, 8, D)[:, :RPT // RCH, :].reshape(NG * NP, 1)
    h1p = _mm1(xx, deg16, W1)
    agg1 = _agg_kernel()(h1p, src, dst, zeros128)
    h2p = _mm2(agg1, h1p, deg16, W2, b1.reshape(1, D))
    agg2 = _agg_kernel()(h2p, src, dst, zeros128)
    G = _mm3(agg2, h2p, deg16, b2.reshape(1, D), fcW.reshape(NG, D, D))
    S = _pair_kernel()(G, y0, y1)
    out = _fc2(S, fcb.reshape(1, D), fc2W, fc2b.reshape(1, 1))
    return out[:P]


# speedup vs baseline: 15.2764x; 1.0076x over previous
import functools

import jax
import jax.numpy as jnp
from jax import lax
from jax.experimental import pallas as pl
from jax.experimental.pallas import tpu as pltpu
from jax.experimental.pallas import tpu_sc as plsc

N = 10000
NP = 10240
NG = 2
E = 320000
E2 = 327680
D = 128
NS = 16
CHUNK = 128
EPT = E2 // NS
NCH = EPT // CHUNK
RPT = NP // NS
RCH = 128
NB = 2
GS = 32
NGR = NCH // GS
P2 = 102400
PPW = P2 // (NG * NS)
PCH = PPW // CHUNK


@functools.cache
def _deg_kernel():
    mesh = plsc.VectorSubcoreMesh(core_axis_name="c", subcore_axis_name="s")
    return pl.kernel(
        _deg_body,
        out_type=jax.ShapeDtypeStruct((NG * NS * 8, D), jnp.float32),
        mesh=mesh,
        scratch_types=[
            pltpu.VMEM((NCH, CHUNK), jnp.int32),
            pltpu.VMEM((RCH, D), jnp.float32),
            pltpu.VMEM((8, D), jnp.float32),
            pltpu.VMEM_SHARED((NP, D), jnp.float32),
            pltpu.SemaphoreType.DMA,
        ],
    )


def _deg_body(dst_hbm, ones_hbm, zeros_hbm, out_hbm,
              didx, zb, zb1, acc, ssem):
    c = lax.axis_index("c")
    s = lax.axis_index("s")
    w = c * NS + s
    pltpu.sync_copy(dst_hbm.at[pl.ds(w * NCH, NCH)], didx)
    pltpu.sync_copy(zeros_hbm, acc.at[pl.ds(s * RPT, RPT)])
    plsc.subcore_barrier()
    pltpu.sync_copy(ones_hbm, zb)

    def fire(j):
        pltpu.async_copy(zb, acc.at[didx.at[j]], ssem, add=True)

    def drain():
        pltpu.make_async_copy(ones_hbm, zb, ssem).wait()

    for j in range(8):
        fire(j)

    def body(i, carry):
        drain()
        fire(i)
        return carry

    lax.fori_loop(8, NCH, body, 0)
    for _ in range(8):
        drain()
    plsc.subcore_barrier()

    lane = lax.iota(jnp.int32, 16)
    for k in range(RPT // RCH):
        pltpu.sync_copy(acc.at[pl.ds(s * RPT + k * RCH, RCH)], zb)

        def extract(i, carry):
            vec = jnp.zeros((16,), jnp.float32)
            for r in range(16):
                row = zb[i * 16 + r, pl.ds(0, 16)]
                vec = jnp.where(lane == r, row, vec)
            zb1[k, pl.ds(i * 16, 16)] = vec
            return carry

        lax.fori_loop(0, RCH // 16, extract, 0)
    pltpu.sync_copy(zb1, out_hbm.at[pl.ds(w * 8, 8)])


@functools.cache
def _agg_kernel():
    mesh = plsc.VectorSubcoreMesh(core_axis_name="c", subcore_axis_name="s")
    return pl.kernel(
        _agg_body,
        out_type=jax.ShapeDtypeStruct((NG * NP, D), jnp.float32),
        mesh=mesh,
        scratch_types=[
            pltpu.VMEM((GS, CHUNK), jnp.int32),
            pltpu.VMEM((GS, CHUNK), jnp.int32),
            pltpu.VMEM((GS, CHUNK), jnp.int32),
            pltpu.VMEM((GS, CHUNK), jnp.int32),
            pltpu.VMEM((NB, CHUNK, D), jnp.float32),
            pltpu.VMEM_SHARED((NP, D), jnp.float32),
            pltpu.SemaphoreType.DMA,
            pltpu.SemaphoreType.DMA,
            pltpu.SemaphoreType.DMA,
        ],
    )


def _agg_body(h_hbm, src_hbm, dst_hbm, zeros_hbm, out_hbm,
              sidx0, didx0, sidx1, didx1, rows, acc, gsem, ssem, isem):
    c = lax.axis_index("c")
    s = lax.axis_index("s")
    w = c * NS + s
    pltpu.sync_copy(zeros_hbm, acc.at[pl.ds(s * RPT, RPT)])
    plsc.subcore_barrier()

    sets = ((sidx0, didx0), (sidx1, didx1))

    def fire_idx(g, st):
        base = w * NCH + g * GS
        pltpu.async_copy(src_hbm.at[pl.ds(base, GS)], st[0], isem)
        pltpu.async_copy(dst_hbm.at[pl.ds(base, GS)], st[1], isem)

    def drain_idx():
        pltpu.make_async_copy(src_hbm.at[pl.ds(0, GS)], sidx0, isem).wait()
        pltpu.make_async_copy(src_hbm.at[pl.ds(0, GS)], didx0, isem).wait()

    def drain(sem):
        pltpu.make_async_copy(h_hbm.at[pl.ds(0, CHUNK)], rows.at[0], sem).wait()

    fire_idx(0, sets[0])
    drain_idx()
    for g in range(NGR):
        sidx, didx = sets[g % 2]
        if g + 1 < NGR:
            fire_idx(g + 1, sets[(g + 1) % 2])

        def fire_g(j, b):
            pltpu.async_copy(h_hbm.at[sidx.at[j]], rows.at[b], gsem)

        def fire_s(j, b):
            pltpu.async_copy(rows.at[b], acc.at[didx.at[j]], ssem, add=True)

        fire_g(0, 0)
        drain(gsem)
        fire_s(0, 0)
        fire_g(1, 1)

        def body(i, carry):
            b = lax.rem(i, 2)
            drain(gsem)
            fire_s(i, b)
            drain(ssem)
            fire_g(i + 1, 1 - b)
            return carry

        lax.fori_loop(1, GS - 1, body, 0)
        drain(gsem)
        fire_s(GS - 1, (GS - 1) % 2)
        drain(ssem)
        drain(ssem)
        if g + 1 < NGR:
            drain_idx()
    plsc.subcore_barrier()
    pltpu.sync_copy(acc.at[pl.ds(s * RPT, RPT)],
                    out_hbm.at[pl.ds(c * NP + s * RPT, RPT)])


@functools.cache
def _pair_kernel():
    mesh = plsc.VectorSubcoreMesh(core_axis_name="c", subcore_axis_name="s")
    return pl.kernel(
        _pair_body,
        out_type=jax.ShapeDtypeStruct((P2, D), jnp.float32),
        mesh=mesh,
        scratch_types=[
            pltpu.VMEM((PCH, CHUNK), jnp.int32),
            pltpu.VMEM((PCH, CHUNK), jnp.int32),
            pltpu.VMEM((3, CHUNK, D), jnp.float32),
            pltpu.VMEM((3, CHUNK, D), jnp.float32),
            pltpu.SemaphoreType.DMA,
            pltpu.SemaphoreType.DMA,
        ],
    )


def _pair_body(g_hbm, y0_hbm, y1_hbm, out_hbm, yi0, yi1, ba, bb, gsem, wsem):
    c = lax.axis_index("c")
    s = lax.axis_index("s")
    w = s * NG + c
    pltpu.sync_copy(y0_hbm.at[w], yi0)
    pltpu.sync_copy(y1_hbm.at[w], yi1)

    def fire_g(j, b):
        pltpu.async_copy(g_hbm.at[yi0.at[j]], ba.at[b], gsem)
        pltpu.async_copy(g_hbm.at[yi1.at[j]], bb.at[b], gsem)

    def drain(sem):
        pltpu.make_async_copy(g_hbm.at[pl.ds(0, CHUNK)], ba.at[0], sem).wait()

    def add_write(j, b):
        def add_row(r, rc):
            for jj in range(D // 16):
                sl = pl.ds(jj * 16, 16)
                ba[b, r, sl] = ba[b, r, sl] + bb[b, r, sl]
            return rc

        lax.fori_loop(0, CHUNK, add_row, 0)
        pltpu.async_copy(ba.at[b],
                         out_hbm.at[pl.ds((w * PCH + j) * CHUNK, CHUNK)], wsem)

    fire_g(0, 0)
    fire_g(1, 1)
    fire_g(2, 2)
    drain(gsem)
    drain(gsem)
    add_write(0, 0)

    def body(j, carry):
        b = lax.rem(j, 3)
        drain(gsem)
        drain(gsem)
        drain(wsem)
        fire_g(j + 2, lax.rem(j + 2, 3))
        add_write(j, b)
        return carry

    lax.fori_loop(1, PCH - 2, body, 0)
    for j in (PCH - 2, PCH - 1):
        drain(gsem)
        drain(gsem)
        drain(wsem)
        add_write(j, j % 3)
    drain(wsem)


_BR = 2048
_BP = 2048

_HI = lax.Precision.HIGHEST


def _mm1_body(x_ref, deg_ref, w_ref, o_ref):
    dis = lax.rsqrt(deg_ref[:, 0] + 1.0)
    h = jnp.dot(x_ref[...], w_ref[...], preferred_element_type=jnp.float32,
                precision=_HI)
    o_ref[...] = h * dis[:, None]


def _mm1(xx, deg16, W1):
    return pl.pallas_call(
        _mm1_body,
        out_shape=jax.ShapeDtypeStruct((NG * NP, D), jnp.float32),
        grid=(NG * NP // _BR,),
        in_specs=[pl.BlockSpec((_BR, D), lambda i: (i, 0)),
                  pl.BlockSpec((_BR, 1), lambda i: (i, 0)),
                  pl.BlockSpec((D, D), lambda i: (0, 0))],
        out_specs=pl.BlockSpec((_BR, D), lambda i: (i, 0)),
    )(xx, deg16, W1)


def _mm2_body(agg_ref, hp_ref, deg_ref, w_ref, b_ref, o_ref):
    dis = lax.rsqrt(deg_ref[:, 0] + 1.0)
    x2 = jnp.maximum((agg_ref[...] + hp_ref[...]) * dis[:, None] + b_ref[...], 0.0)
    h = jnp.dot(x2, w_ref[...], preferred_element_type=jnp.float32, precision=_HI)
    o_ref[...] = h * dis[:, None]


def _mm2(agg, hp, deg16, W2, b1):
    return pl.pallas_call(
        _mm2_body,
        out_shape=jax.ShapeDtypeStruct((NG * NP, D), jnp.float32),
        grid=(NG * NP // _BR,),
        in_specs=[pl.BlockSpec((_BR, D), lambda i: (i, 0)),
                  pl.BlockSpec((_BR, D), lambda i: (i, 0)),
                  pl.BlockSpec((_BR, 1), lambda i: (i, 0)),
                  pl.BlockSpec((D, D), lambda i: (0, 0)),
                  pl.BlockSpec((1, D), lambda i: (0, 0))],
        out_specs=pl.BlockSpec((_BR, D), lambda i: (i, 0)),
    )(agg, hp, deg16, W2, b1)


def _mm3_body(agg_ref, hp_ref, deg_ref, b_ref, fw_ref, o_ref):
    dis = lax.rsqrt(deg_ref[:, 0] + 1.0)
    xf = jnp.maximum((agg_ref[...] + hp_ref[...]) * dis[:, None] + b_ref[...], 0.0)
    o_ref[...] = jnp.dot(xf, fw_ref[0], preferred_element_type=jnp.float32,
                         precision=_HI)


def _mm3(agg, hp, deg16, b2, fcW3):
    nblk = NG * NP // _BR
    half = nblk // NG
    return pl.pallas_call(
        _mm3_body,
        out_shape=jax.ShapeDtypeStruct((NG * NP, D), jnp.float32),
        grid=(nblk,),
        in_specs=[pl.BlockSpec((_BR, D), lambda i: (i, 0)),
                  pl.BlockSpec((_BR, D), lambda i: (i, 0)),
                  pl.BlockSpec((_BR, 1), lambda i: (i, 0)),
                  pl.BlockSpec((1, D), lambda i: (0, 0)),
                  pl.BlockSpec((1, D, D), lambda i: (i // half, 0, 0))],
        out_specs=pl.BlockSpec((_BR, D), lambda i: (i, 0)),
    )(agg, hp, deg16, b2, fcW3)


def _fc2_body(s_ref, fcb_ref, w_ref, b2_ref, o_ref):
    z = jnp.maximum(s_ref[...] + fcb_ref[...], 0.0)
    o = jnp.dot(z, w_ref[...], preferred_element_type=jnp.float32,
                precision=_HI) + b2_ref[...]
    o_ref[...] = 1.0 / (1.0 + jnp.exp(-o))


def _fc2(S, fcb, fc2W, fc2b):
    return pl.pallas_call(
        _fc2_body,
        out_shape=jax.ShapeDtypeStruct((P2, 1), jnp.float32),
        grid=(P2 // _BP,),
        in_specs=[pl.BlockSpec((_BP, D), lambda i: (i, 0)),
                  pl.BlockSpec((1, D), lambda i: (0, 0)),
                  pl.BlockSpec((D, 1), lambda i: (0, 0)),
                  pl.BlockSpec((1, 1), lambda i: (0, 0))],
        out_specs=pl.BlockSpec((_BP, 1), lambda i: (i, 0)),
    )(S, fcb, fc2W, fc2b)


def kernel(x_s, edge_index_s, x_t, edge_index_t, y, W1, b1, W2, b2,
           fcW, fcb, fc2W, fc2b):
    P = y.shape[0]
    zpad = jnp.zeros((NP - N, D), jnp.float32)
    xx = jnp.concatenate([x_s, zpad, x_t, zpad], axis=0)

    epad = E2 - E
    pad_src = (jnp.arange(epad, dtype=jnp.int32) * 37) % N
    pad_dst = N + (jnp.arange(epad, dtype=jnp.int32) % (NP - N))
    src = jnp.concatenate([edge_index_s[0].astype(jnp.int32), pad_src,
                           edge_index_t[0].astype(jnp.int32) + NP,
                           pad_src + NP]).reshape(NG * NS * NCH, CHUNK)
    dst = jnp.concatenate([edge_index_s[1].astype(jnp.int32), pad_dst,
                           edge_index_t[1].astype(jnp.int32),
                           pad_dst]).reshape(NG * NS * NCH, CHUNK)

    pad = P2 - P
    pad_y = (jnp.arange(pad, dtype=jnp.int32) * 53) % N
    y0 = jnp.concatenate([y[:, 0].astype(jnp.int32),
                          pad_y]).reshape(NG * NS, PCH, CHUNK)
    y1 = jnp.concatenate([y[:, 1].astype(jnp.int32) + NP,
                          pad_y + NP]).reshape(NG * NS, PCH, CHUNK)

    ones128 = jnp.ones((CHUNK, D), jnp.float32)
    zeros128 = jnp.zeros((RPT, D), jnp.float32)

    degp = _deg_kernel()(dst, ones128, zeros128)
    deg16 = degp.reshape(NG * NS, 8, D)[:, :RPT // RCH, :].reshape(NG * NP, 1)
    h1p = _mm1(xx, deg16, W1)
    agg1 = _agg_kernel()(h1p, src, dst, zeros128)
    h2p = _mm2(agg1, h1p, deg16, W2, b1.reshape(1, D))
    agg2 = _agg_kernel()(h2p, src, dst, zeros128)
    G = _mm3(agg2, h2p, deg16, b2.reshape(1, D), fcW.reshape(NG, D, D))
    S = _pair_kernel()(G, y0, y1)
    out = _fc2(S, fcb.reshape(1, D), fc2W, fc2b.reshape(1, 1))
    return out[:P]
